# baseline (speedup 1.0000x reference)
import jax
import jax.numpy as jnp
from jax import lax
from jax.experimental import pallas as pl
from jax.experimental.pallas import tpu as pltpu
from jax.experimental.pallas import tpu_sc as plsc

N = 10000
D = 128
E = 320000

NPAD = 10240
B = 128
NB = 160
CH = 16
NCK = NB // CH
ECH = NB * B
EPAD = 16 * ECH
NCHUNK = NPAD // 16
DH = D // 2
DW = DH + 16


def _ab_body(q_ref, w2_ref, out_ref):
    out_ref[...] = jnp.dot(q_ref[...], w2_ref[...],
                           preferred_element_type=jnp.float32)


def _ab_call(qpad, w2):
    return pl.pallas_call(
        _ab_body,
        out_shape=jax.ShapeDtypeStruct((NPAD, 2), jnp.float32),
    )(qpad, w2)


def _sc_body(qh_hbm, edges_hbm, ab_hbm, out_hbm,
             alpha_t, beta_t, idx_in_t, idx_out_t, ex_t, rows_t, rows_a_t,
             u_s, gsem, ssem, iisem, iosem):
    c = lax.axis_index("c")
    s = lax.axis_index("s")

    pltpu.sync_copy(ab_hbm.at[0], alpha_t)
    pltpu.sync_copy(ab_hbm.at[1], beta_t)

    eic = edges_hbm.at[c].at[s]
    eoc = edges_hbm.at[1 - c].at[s]

    zeros16 = jnp.zeros((16,), jnp.float32)
    lane0 = lax.iota(jnp.int32, 16) == 0

    def _zero_row(r, carry):
        for k in range(DW // 16):
            rows_a_t[0, r, pl.ds(16 * k, 16)] = zeros16
        return carry

    def _zero_u():
        lax.fori_loop(0, B, _zero_row, 0)
        pltpu.sync_copy(rows_a_t.at[0], u_s.at[pl.ds(0, B)])

    _zero_u()
    plsc.subcore_barrier()

    def _phase(p, carry0):
        qt = qh_hbm.at[p]
        pltpu.sync_copy(eic.at[pl.ds(0, CH)], idx_in_t.at[0])
        pltpu.sync_copy(eoc.at[pl.ds(0, CH)], idx_out_t.at[0])
        for jj2 in (0, 1, 2):
            pltpu.async_copy(qt.at[idx_in_t.at[0].at[jj2]], rows_t.at[jj2],
                             gsem.at[jj2])

        def _chunk(g, carry):
            sub = lax.rem(g, 2)
            nxt = 1 - sub

            @pl.when(g > 0)
            def _():
                pltpu.make_async_copy(eic.at[pl.ds(g * CH, CH)],
                                      idx_in_t.at[sub], iisem).wait()
                pltpu.make_async_copy(eoc.at[pl.ds(g * CH, CH)],
                                      idx_out_t.at[sub], iosem).wait()
                for jj2 in (0, 1, 2):
                    pltpu.async_copy(qt.at[idx_in_t.at[sub].at[jj2]],
                                     rows_t.at[jj2], gsem.at[jj2])

            @pl.when(g + 1 < NCK)
            def _():
                pltpu.async_copy(eic.at[pl.ds((g + 1) * CH, CH)],
                                 idx_in_t.at[nxt], iisem)
                pltpu.async_copy(eoc.at[pl.ds((g + 1) * CH, CH)],
                                 idx_out_t.at[nxt], iosem)

            for jj in range(CH):
                buf = jj % 3
                pltpu.make_async_copy(qt.at[idx_in_t.at[sub].at[jj]],
                                      rows_t.at[buf], gsem.at[buf]).wait()
                pass

                pass

                def _scale_row(r, carry2):
                    wv = plsc.load_gather(
                        ex_t, [jnp.full((16,), r, jnp.int32)])
                    for k in range(DH // 16):
                        rows_a_t[buf, r, pl.ds(16 * k, 16)] = (
                            rows_t[buf, r, pl.ds(16 * k, 16)] * wv)
                    rows_a_t[buf, r, pl.ds(DH, 16)] = jnp.where(
                        lane0, wv, zeros16)
                    return carry2
                pass

                pass
                if jj < CH - 3:
                    pltpu.async_copy(qt.at[idx_in_t.at[sub].at[jj + 3]],
                                     rows_t.at[buf], gsem.at[buf])
            return carry

        lax.fori_loop(0, NCK, _chunk, 0)

        pass
        plsc.subcore_barrier()

        for i in range(NCHUNK // B):
            off = s * NCHUNK + i * B
            pltpu.sync_copy(u_s.at[pl.ds(0, B)],
                            out_hbm.at[c].at[p].at[pl.ds(off, B)])

        @pl.when(p == 0)
        def _():
            _zero_u()
        plsc.subcore_barrier()
        return carry0

    lax.fori_loop(0, 2, _phase, 0)


_sc_call = pl.kernel(
    _sc_body,
    out_type=jax.ShapeDtypeStruct((2, 2, NPAD, DW), jnp.float32),
    mesh=plsc.VectorSubcoreMesh(core_axis_name="c", subcore_axis_name="s",
                                num_cores=2, num_subcores=16),
    compiler_params=pltpu.CompilerParams(needs_layout_passes=False,
                                         use_tc_tiling_on_sc=False),
    scratch_types=[
        pltpu.VMEM((NPAD,), jnp.float32),
        pltpu.VMEM((NPAD,), jnp.float32),
        pltpu.VMEM((2, CH, B), jnp.int32),
        pltpu.VMEM((2, CH, B), jnp.int32),
        pltpu.VMEM((B,), jnp.float32),
        pltpu.VMEM((3, B, D), jnp.float32),
        pltpu.VMEM((2, B, DW), jnp.float32),
        pltpu.VMEM_SHARED((2048, DW), jnp.float32),
        pltpu.SemaphoreType.DMA((3,)),
        pltpu.SemaphoreType.DMA((2,)),
        pltpu.SemaphoreType.DMA,
        pltpu.SemaphoreType.DMA,
    ],
)


def _out_body(q_ref, a00_ref, a01_ref, a10_ref, a11_ref,
              wu_ref, wv_ref, bv_ref, o1_ref, o2_ref):
    a00 = a00_ref[...]
    a01 = a01_ref[...]
    a10 = a10_ref[...]
    a11 = a11_ref[...]
    u1 = jnp.concatenate([a00[:, :DH], a01[:, :DH]], axis=1)
    u2 = jnp.concatenate([a10[:, :DH], a11[:, :DH]], axis=1)
    s1 = a00[:, DH:DH + 1]
    s2 = a10[:, DH:DH + 1]
    inv1 = jnp.where(s1 > 0.0, 1.0 / s1, 0.0)
    inv2 = jnp.where(s2 > 0.0, 1.0 / s2, 0.0)
    ok = u1 * inv1 + u2 * inv2
    q = q_ref[...]
    o1_ref[...] = jnp.dot(2.0 * q + ok, wu_ref[...],
                          preferred_element_type=jnp.float32)
    cnt = ((s1 > 0.0).astype(jnp.float32) + (s2 > 0.0).astype(jnp.float32))
    o2_ref[...] = (jnp.dot(ok, wv_ref[...],
                           preferred_element_type=jnp.float32)
                   + cnt * bv_ref[...])


def _out_call(qpad, u, W_upd, W_V, bv):
    R = 1024
    grid = (NPAD // R,)
    row_spec = pl.BlockSpec((R, D), lambda i: (i, 0))
    acc_spec = pl.BlockSpec((R, DW), lambda i: (i, 0))
    full_spec = pl.BlockSpec((D, D), lambda i: (0, 0))
    bv_spec = pl.BlockSpec((1, D), lambda i: (0, 0))
    return pl.pallas_call(
        _out_body,
        grid=grid,
        in_specs=[row_spec, acc_spec, acc_spec, acc_spec, acc_spec,
                  full_spec, full_spec, bv_spec],
        out_specs=[row_spec, row_spec],
        out_shape=[jax.ShapeDtypeStruct((NPAD, D), jnp.float32),
                   jax.ShapeDtypeStruct((NPAD, D), jnp.float32)],
    )(qpad, u[0, 0], u[0, 1], u[1, 0], u[1, 1], W_upd, W_V, bv)


@jax.jit
def kernel(q, edge_index, W_att, W_upd, W_V, b_V):
    src = edge_index[0]
    dst = edge_index[1]
    pads = EPAD - E
    srcp = jnp.pad(src, (0, pads), constant_values=N).reshape(16, NB, B)
    dstp = jnp.pad(dst, (0, pads), constant_values=N).reshape(16, NB, B)
    edges3 = jnp.stack([srcp, dstp])
    qpad = jnp.pad(q, ((0, NPAD - N), (0, 0)))
    w2 = jnp.stack([W_att[:D, 0], W_att[D:, 0]], axis=1)

    qh = jnp.stack([qpad, qpad])
    ab = _ab_call(qpad, w2).T
    u = _sc_call(qh, edges3, ab)
    o1, o2 = _out_call(qpad, u, W_upd, W_V, b_V.reshape(1, D))
    return o1[:N], o2[:N]


# speedup vs baseline: 1.7446x; 1.7446x over previous
import jax
import jax.numpy as jnp
from jax import lax
from jax.experimental import pallas as pl
from jax.experimental.pallas import tpu as pltpu
from jax.experimental.pallas import tpu_sc as plsc

N = 10000
D = 128
E = 320000

NPAD = 10240
B = 128
NB = 160
CH = 16
NCK = NB // CH
ECH = NB * B
EPAD = 16 * ECH
NCHUNK = NPAD // 16
DH = D // 2
DW = DH + 16


def _ab_body(q_ref, w2_ref, out_ref):
    out_ref[...] = jnp.dot(q_ref[...], w2_ref[...],
                           preferred_element_type=jnp.float32)


def _ab_call(qpad, w2):
    return pl.pallas_call(
        _ab_body,
        out_shape=jax.ShapeDtypeStruct((NPAD, 2), jnp.float32),
    )(qpad, w2)


def _sc_body(qh_hbm, edges_hbm, ab_hbm, out_hbm,
             alpha_t, beta_t, idx_in_t, idx_out_t, ex_t, rows_t, rows_a_t,
             u_s, gsem, ssem, iisem, iosem):
    c = lax.axis_index("c")
    s = lax.axis_index("s")

    pltpu.sync_copy(ab_hbm.at[0], alpha_t)
    pltpu.sync_copy(ab_hbm.at[1], beta_t)

    eic = edges_hbm.at[c].at[s]
    eoc = edges_hbm.at[1 - c].at[s]

    zeros16 = jnp.zeros((16,), jnp.float32)
    lane0 = lax.iota(jnp.int32, 16) == 0

    def _zero_row(r, carry):
        for k in range(DW // 16):
            rows_a_t[0, r, pl.ds(16 * k, 16)] = zeros16
        return carry

    def _zero_u():
        lax.fori_loop(0, B, _zero_row, 0)
        for i in range(NCHUNK // B):
            pltpu.sync_copy(rows_a_t.at[0],
                            u_s.at[pl.ds(s * NCHUNK + i * B, B)])

    _zero_u()
    plsc.subcore_barrier()

    def _phase(p, carry0):
        qt = qh_hbm.at[p]
        pltpu.sync_copy(eic.at[pl.ds(0, CH)], idx_in_t.at[0])
        pltpu.sync_copy(eoc.at[pl.ds(0, CH)], idx_out_t.at[0])
        for jj2 in (0, 1):
            pltpu.async_copy(qt.at[idx_in_t.at[0].at[jj2]], rows_t.at[jj2],
                             gsem.at[jj2])

        def _chunk(g, carry):
            sub = lax.rem(g, 2)
            nxt = 1 - sub

            @pl.when(g > 0)
            def _():
                pltpu.make_async_copy(eic.at[pl.ds(g * CH, CH)],
                                      idx_in_t.at[sub], iisem).wait()
                pltpu.make_async_copy(eoc.at[pl.ds(g * CH, CH)],
                                      idx_out_t.at[sub], iosem).wait()
                for jj2 in (0, 1):
                    pltpu.async_copy(qt.at[idx_in_t.at[sub].at[jj2]],
                                     rows_t.at[jj2], gsem.at[jj2])

            @pl.when(g + 1 < NCK)
            def _():
                pltpu.async_copy(eic.at[pl.ds((g + 1) * CH, CH)],
                                 idx_in_t.at[nxt], iisem)
                pltpu.async_copy(eoc.at[pl.ds((g + 1) * CH, CH)],
                                 idx_out_t.at[nxt], iosem)

            for jj in range(CH):
                buf = jj % 2
                pltpu.make_async_copy(qt.at[idx_in_t.at[sub].at[jj]],
                                      rows_t.at[buf], gsem.at[buf]).wait()
                for k in range(B // 16):
                    vi = idx_in_t[sub, jj, pl.ds(16 * k, 16)]
                    vo = idx_out_t[sub, jj, pl.ds(16 * k, 16)]
                    a = plsc.load_gather(alpha_t, [vi])
                    b = plsc.load_gather(beta_t, [vo])
                    sg = 1.0 / (1.0 + jnp.exp(-(a + b)))
                    ex_t[pl.ds(16 * k, 16)] = jnp.exp(sg)

                def _drain_scatter():
                    pltpu.make_async_copy(
                        rows_a_t.at[buf],
                        u_s.at[idx_out_t.at[sub].at[jj]],
                        ssem.at[buf]).wait()
                if jj >= 2:
                    _drain_scatter()
                else:
                    @pl.when(g > 0)
                    def _():
                        _drain_scatter()

                def _scale_row(r, carry2):
                    wv = plsc.load_gather(
                        ex_t, [jnp.full((16,), r, jnp.int32)])
                    for k in range(DH // 32):
                        x32 = rows_t[buf, r, pl.ds(32 * k, 32)]
                        lo, hi = plsc.unpack(x32, format=plsc.PackFormat.INTERLEAVED)
                        rows_a_t[buf, r, pl.ds(32 * k, 16)] = lo * wv
                        rows_a_t[buf, r, pl.ds(32 * k + 16, 16)] = hi * wv
                    rows_a_t[buf, r, pl.ds(DH, 16)] = jnp.where(
                        lane0, wv, zeros16)
                    return carry2
                lax.fori_loop(0, B, _scale_row, 0)

                pltpu.async_copy(rows_a_t.at[buf],
                                 u_s.at[idx_out_t.at[sub].at[jj]],
                                 ssem.at[buf], add=True)
                if jj < CH - 2:
                    pltpu.async_copy(qt.at[idx_in_t.at[sub].at[jj + 2]],
                                     rows_t.at[buf], gsem.at[buf])
            return carry

        lax.fori_loop(0, NCK, _chunk, 0)

        for buf in (0, 1):
            pltpu.make_async_copy(rows_a_t.at[buf],
                                  u_s.at[idx_out_t.at[1].at[CH - 2 + buf]],
                                  ssem.at[buf]).wait()
        plsc.subcore_barrier()

        for i in range(NCHUNK // B):
            off = s * NCHUNK + i * B
            pltpu.sync_copy(u_s.at[pl.ds(off, B)],
                            out_hbm.at[c].at[p].at[pl.ds(off, B)])

        @pl.when(p == 0)
        def _():
            _zero_u()
        plsc.subcore_barrier()
        return carry0

    lax.fori_loop(0, 2, _phase, 0)


_sc_call = pl.kernel(
    _sc_body,
    out_type=jax.ShapeDtypeStruct((2, 2, NPAD, DW), jnp.float32),
    mesh=plsc.VectorSubcoreMesh(core_axis_name="c", subcore_axis_name="s",
                                num_cores=2, num_subcores=16),
    compiler_params=pltpu.CompilerParams(needs_layout_passes=False,
                                         use_tc_tiling_on_sc=False),
    scratch_types=[
        pltpu.VMEM((NPAD,), jnp.float32),
        pltpu.VMEM((NPAD,), jnp.float32),
        pltpu.VMEM((2, CH, B), jnp.int32),
        pltpu.VMEM((2, CH, B), jnp.int32),
        pltpu.VMEM((B,), jnp.float32),
        pltpu.VMEM((2, B, DH), jnp.bfloat16),
        pltpu.VMEM((2, B, DW), jnp.float32),
        pltpu.VMEM_SHARED((NPAD, DW), jnp.float32),
        pltpu.SemaphoreType.DMA((2,)),
        pltpu.SemaphoreType.DMA((2,)),
        pltpu.SemaphoreType.DMA,
        pltpu.SemaphoreType.DMA,
    ],
)


def _out_body(q_ref, a00_ref, a01_ref, a10_ref, a11_ref,
              wu_ref, wv_ref, bv_ref, o1_ref, o2_ref):
    a00 = a00_ref[...]
    a01 = a01_ref[...]
    a10 = a10_ref[...]
    a11 = a11_ref[...]
    u1 = jnp.concatenate([a00[:, :DH], a01[:, :DH]], axis=1)
    u2 = jnp.concatenate([a10[:, :DH], a11[:, :DH]], axis=1)
    s1 = a00[:, DH:DH + 1]
    s2 = a10[:, DH:DH + 1]
    inv1 = jnp.where(s1 > 0.0, 1.0 / s1, 0.0)
    inv2 = jnp.where(s2 > 0.0, 1.0 / s2, 0.0)
    ok = u1 * inv1 + u2 * inv2
    q = q_ref[...]
    o1_ref[...] = jnp.dot(2.0 * q + ok, wu_ref[...],
                          preferred_element_type=jnp.float32)
    cnt = ((s1 > 0.0).astype(jnp.float32) + (s2 > 0.0).astype(jnp.float32))
    o2_ref[...] = (jnp.dot(ok, wv_ref[...],
                           preferred_element_type=jnp.float32)
                   + cnt * bv_ref[...])


def _out_call(qpad, u, W_upd, W_V, bv):
    R = 1024
    grid = (NPAD // R,)
    row_spec = pl.BlockSpec((R, D), lambda i: (i, 0))
    acc_spec = pl.BlockSpec((R, DW), lambda i: (i, 0))
    full_spec = pl.BlockSpec((D, D), lambda i: (0, 0))
    bv_spec = pl.BlockSpec((1, D), lambda i: (0, 0))
    return pl.pallas_call(
        _out_body,
        grid=grid,
        in_specs=[row_spec, acc_spec, acc_spec, acc_spec, acc_spec,
                  full_spec, full_spec, bv_spec],
        out_specs=[row_spec, row_spec],
        out_shape=[jax.ShapeDtypeStruct((NPAD, D), jnp.float32),
                   jax.ShapeDtypeStruct((NPAD, D), jnp.float32)],
    )(qpad, u[0, 0], u[0, 1], u[1, 0], u[1, 1], W_upd, W_V, bv)


def _make_qperm():
    perm = []
    for blk in range(D // 32):
        base = blk * 32
        for i in range(16):
            perm.append(base + i)
            perm.append(base + 16 + i)
    return tuple(perm)


_QPERM = _make_qperm()


@jax.jit
def kernel(q, edge_index, W_att, W_upd, W_V, b_V):
    src = edge_index[0]
    dst = edge_index[1]
    pads = EPAD - E
    srcp = jnp.pad(src, (0, pads), constant_values=N).reshape(16, NB, B)
    dstp = jnp.pad(dst, (0, pads), constant_values=N).reshape(16, NB, B)
    edges3 = jnp.stack([srcp, dstp])
    qpad = jnp.pad(q, ((0, NPAD - N), (0, 0)))
    w2 = jnp.stack([W_att[:D, 0], W_att[D:, 0]], axis=1)

    qb = qpad.astype(jnp.bfloat16)[:, _QPERM]
    qh = jnp.stack([qb[:, :DH], qb[:, DH:]])
    ab = _ab_call(qpad, w2).T
    u = _sc_call(qh, edges3, ab)
    o1, o2 = _out_call(qpad, u, W_upd, W_V, b_V.reshape(1, D))
    return o1[:N], o2[:N]


# speedup vs baseline: 2.4226x; 1.3886x over previous
import jax
import jax.numpy as jnp
from jax import lax
from jax.experimental import pallas as pl
from jax.experimental.pallas import tpu as pltpu
from jax.experimental.pallas import tpu_sc as plsc

N = 10000
D = 128
E = 320000

NPAD = 10240
B = 128
NB = 160
CH = 16
NCK = NB // CH
ECH = NB * B
EPAD = 16 * ECH
NCHUNK = NPAD // 16
DH = D // 2
DW = DH + 16


def _ab_body(q_ref, w2_ref, out_ref):
    out_ref[...] = jnp.dot(q_ref[...], w2_ref[...],
                           preferred_element_type=jnp.float32)


def _ab_call(qpad, w2):
    return pl.pallas_call(
        _ab_body,
        out_shape=jax.ShapeDtypeStruct((NPAD, 2), jnp.float32),
    )(qpad, w2)


def _sc_body(qh_hbm, edges_hbm, ab_hbm, out_hbm,
             alpha_t, beta_t, idx_in_t, idx_out_t, ex_t, rows_t, rows_a_t,
             u_s, gsem, ssem, iisem, iosem):
    c = lax.axis_index("c")
    s = lax.axis_index("s")

    pltpu.sync_copy(ab_hbm.at[0], alpha_t)
    pltpu.sync_copy(ab_hbm.at[1], beta_t)

    eic = edges_hbm.at[c].at[s]
    eoc = edges_hbm.at[1 - c].at[s]

    zeros16 = jnp.zeros((16,), jnp.float32)
    lane0 = lax.iota(jnp.int32, 16) == 0

    def _zero_row(r, carry):
        for k in range(DW // 16):
            rows_a_t[0, r, pl.ds(16 * k, 16)] = zeros16
        return carry

    def _zero_u():
        lax.fori_loop(0, B, _zero_row, 0)
        for i in range(NCHUNK // B):
            pltpu.sync_copy(rows_a_t.at[0],
                            u_s.at[pl.ds(s * NCHUNK + i * B, B)])

    _zero_u()
    plsc.subcore_barrier()

    def _phase(p, carry0):
        qt = qh_hbm.at[p]
        pltpu.sync_copy(eic.at[pl.ds(0, CH)], idx_in_t.at[0])
        pltpu.sync_copy(eoc.at[pl.ds(0, CH)], idx_out_t.at[0])
        for jj2 in (0, 1):
            pltpu.async_copy(qt.at[idx_in_t.at[0].at[jj2]], rows_t.at[jj2],
                             gsem.at[jj2])

        def _chunk(g, carry):
            sub = lax.rem(g, 2)
            nxt = 1 - sub

            @pl.when(g > 0)
            def _():
                pltpu.make_async_copy(eic.at[pl.ds(g * CH, CH)],
                                      idx_in_t.at[sub], iisem).wait()
                pltpu.make_async_copy(eoc.at[pl.ds(g * CH, CH)],
                                      idx_out_t.at[sub], iosem).wait()
                for jj2 in (0, 1):
                    pltpu.async_copy(qt.at[idx_in_t.at[sub].at[jj2]],
                                     rows_t.at[jj2], gsem.at[jj2])

            @pl.when(g + 1 < NCK)
            def _():
                pltpu.async_copy(eic.at[pl.ds((g + 1) * CH, CH)],
                                 idx_in_t.at[nxt], iisem)
                pltpu.async_copy(eoc.at[pl.ds((g + 1) * CH, CH)],
                                 idx_out_t.at[nxt], iosem)

            for jj in range(CH):
                buf = jj % 2
                pltpu.make_async_copy(qt.at[idx_in_t.at[sub].at[jj]],
                                      rows_t.at[buf], gsem.at[buf]).wait()
                for k in range(B // 16):
                    vi = idx_in_t[sub, jj, pl.ds(16 * k, 16)]
                    vo = idx_out_t[sub, jj, pl.ds(16 * k, 16)]
                    a = plsc.load_gather(alpha_t, [vi])
                    b = plsc.load_gather(beta_t, [vo])
                    sg = 1.0 / (1.0 + jnp.exp(-(a + b)))
                    ex_t[pl.ds(16 * k, 16)] = jnp.exp(sg)

                def _drain_scatter():
                    pltpu.make_async_copy(
                        rows_a_t.at[buf],
                        u_s.at[idx_out_t.at[sub].at[jj]],
                        ssem.at[buf]).wait()
                if jj >= 2:
                    _drain_scatter()
                else:
                    @pl.when(g > 0)
                    def _():
                        _drain_scatter()

                @plsc.parallel_loop(0, B, step=1, unroll=4)
                def _scale_row(r):
                    wv = plsc.load_gather(
                        ex_t, [jnp.full((16,), r, jnp.int32)])
                    for k in range(DH // 32):
                        x32 = rows_t[buf, r, pl.ds(32 * k, 32)]
                        lo, hi = plsc.unpack(x32, format=plsc.PackFormat.INTERLEAVED)
                        rows_a_t[buf, r, pl.ds(32 * k, 16)] = lo * wv
                        rows_a_t[buf, r, pl.ds(32 * k + 16, 16)] = hi * wv
                    rows_a_t[buf, r, pl.ds(DH, 16)] = jnp.where(
                        lane0, wv, zeros16)

                pltpu.async_copy(rows_a_t.at[buf],
                                 u_s.at[idx_out_t.at[sub].at[jj]],
                                 ssem.at[buf], add=True)
                if jj < CH - 2:
                    pltpu.async_copy(qt.at[idx_in_t.at[sub].at[jj + 2]],
                                     rows_t.at[buf], gsem.at[buf])
            return carry

        lax.fori_loop(0, NCK, _chunk, 0)

        for buf in (0, 1):
            pltpu.make_async_copy(rows_a_t.at[buf],
                                  u_s.at[idx_out_t.at[1].at[CH - 2 + buf]],
                                  ssem.at[buf]).wait()
        plsc.subcore_barrier()

        for i in range(NCHUNK // B):
            off = s * NCHUNK + i * B
            pltpu.sync_copy(u_s.at[pl.ds(off, B)],
                            out_hbm.at[c].at[p].at[pl.ds(off, B)])

        @pl.when(p == 0)
        def _():
            _zero_u()
        plsc.subcore_barrier()
        return carry0

    lax.fori_loop(0, 2, _phase, 0)


_sc_call = pl.kernel(
    _sc_body,
    out_type=jax.ShapeDtypeStruct((2, 2, NPAD, DW), jnp.float32),
    mesh=plsc.VectorSubcoreMesh(core_axis_name="c", subcore_axis_name="s",
                                num_cores=2, num_subcores=16),
    compiler_params=pltpu.CompilerParams(needs_layout_passes=False,
                                         use_tc_tiling_on_sc=False),
    scratch_types=[
        pltpu.VMEM((NPAD,), jnp.float32),
        pltpu.VMEM((NPAD,), jnp.float32),
        pltpu.VMEM((2, CH, B), jnp.int32),
        pltpu.VMEM((2, CH, B), jnp.int32),
        pltpu.VMEM((B,), jnp.float32),
        pltpu.VMEM((2, B, DH), jnp.bfloat16),
        pltpu.VMEM((2, B, DW), jnp.float32),
        pltpu.VMEM_SHARED((NPAD, DW), jnp.float32),
        pltpu.SemaphoreType.DMA((2,)),
        pltpu.SemaphoreType.DMA((2,)),
        pltpu.SemaphoreType.DMA,
        pltpu.SemaphoreType.DMA,
    ],
)


def _out_body(q_ref, a00_ref, a01_ref, a10_ref, a11_ref,
              wu_ref, wv_ref, bv_ref, o1_ref, o2_ref):
    a00 = a00_ref[...]
    a01 = a01_ref[...]
    a10 = a10_ref[...]
    a11 = a11_ref[...]
    u1 = jnp.concatenate([a00[:, :DH], a01[:, :DH]], axis=1)
    u2 = jnp.concatenate([a10[:, :DH], a11[:, :DH]], axis=1)
    s1 = a00[:, DH:DH + 1]
    s2 = a10[:, DH:DH + 1]
    inv1 = jnp.where(s1 > 0.0, 1.0 / s1, 0.0)
    inv2 = jnp.where(s2 > 0.0, 1.0 / s2, 0.0)
    ok = u1 * inv1 + u2 * inv2
    q = q_ref[...]
    o1_ref[...] = jnp.dot(2.0 * q + ok, wu_ref[...],
                          preferred_element_type=jnp.float32)
    cnt = ((s1 > 0.0).astype(jnp.float32) + (s2 > 0.0).astype(jnp.float32))
    o2_ref[...] = (jnp.dot(ok, wv_ref[...],
                           preferred_element_type=jnp.float32)
                   + cnt * bv_ref[...])


def _out_call(qpad, u, W_upd, W_V, bv):
    R = 1024
    grid = (NPAD // R,)
    row_spec = pl.BlockSpec((R, D), lambda i: (i, 0))
    acc_spec = pl.BlockSpec((R, DW), lambda i: (i, 0))
    full_spec = pl.BlockSpec((D, D), lambda i: (0, 0))
    bv_spec = pl.BlockSpec((1, D), lambda i: (0, 0))
    return pl.pallas_call(
        _out_body,
        grid=grid,
        in_specs=[row_spec, acc_spec, acc_spec, acc_spec, acc_spec,
                  full_spec, full_spec, bv_spec],
        out_specs=[row_spec, row_spec],
        out_shape=[jax.ShapeDtypeStruct((NPAD, D), jnp.float32),
                   jax.ShapeDtypeStruct((NPAD, D), jnp.float32)],
    )(qpad, u[0, 0], u[0, 1], u[1, 0], u[1, 1], W_upd, W_V, bv)


def _make_qperm():
    perm = []
    for blk in range(D // 32):
        base = blk * 32
        for i in range(16):
            perm.append(base + i)
            perm.append(base + 16 + i)
    return tuple(perm)


_QPERM = _make_qperm()


@jax.jit
def kernel(q, edge_index, W_att, W_upd, W_V, b_V):
    src = edge_index[0]
    dst = edge_index[1]
    pads = EPAD - E
    srcp = jnp.pad(src, (0, pads), constant_values=N).reshape(16, NB, B)
    dstp = jnp.pad(dst, (0, pads), constant_values=N).reshape(16, NB, B)
    edges3 = jnp.stack([srcp, dstp])
    qpad = jnp.pad(q, ((0, NPAD - N), (0, 0)))
    w2 = jnp.stack([W_att[:D, 0], W_att[D:, 0]], axis=1)

    qb = qpad.astype(jnp.bfloat16)[:, _QPERM]
    qh = jnp.stack([qb[:, :DH], qb[:, DH:]])
    ab = _ab_call(qpad, w2).T
    u = _sc_call(qh, edges3, ab)
    o1, o2 = _out_call(qpad, u, W_upd, W_V, b_V.reshape(1, D))
    return o1[:N], o2[:N]


# speedup vs baseline: 2.4997x; 1.0318x over previous
import jax
import jax.numpy as jnp
from jax import lax
from jax.experimental import pallas as pl
from jax.experimental.pallas import tpu as pltpu
from jax.experimental.pallas import tpu_sc as plsc

N = 10000
D = 128
E = 320000

NPAD = 10240
B = 128
NB = 160
CH = 16
NCK = NB // CH
ECH = NB * B
EPAD = 16 * ECH
NCHUNK = NPAD // 16
DH = D // 2
DW = DH + 16


def _prep_body(q_ref, w2_ref, pm_ref, ab_ref, qh_ref):
    q = q_ref[...]
    ab_ref[...] = jnp.dot(q, w2_ref[...], preferred_element_type=jnp.float32)
    qp = jnp.dot(q, pm_ref[...], preferred_element_type=jnp.float32)
    qb = qp.astype(jnp.bfloat16)
    qh_ref[0] = qb[:, :DH]
    qh_ref[1] = qb[:, DH:]


def _prep_call(qpad, w2, pm):
    R = 1024
    return pl.pallas_call(
        _prep_body,
        grid=(NPAD // R,),
        in_specs=[pl.BlockSpec((R, D), lambda i: (i, 0)),
                  pl.BlockSpec((D, 2), lambda i: (0, 0)),
                  pl.BlockSpec((D, D), lambda i: (0, 0))],
        out_specs=[pl.BlockSpec((R, 2), lambda i: (i, 0)),
                   pl.BlockSpec((2, R, DH), lambda i: (0, i, 0))],
        out_shape=[jax.ShapeDtypeStruct((NPAD, 2), jnp.float32),
                   jax.ShapeDtypeStruct((2, NPAD, DH), jnp.bfloat16)],
    )(qpad, w2, pm)


def _sc_body(qh_hbm, edges_hbm, ab_hbm, out_hbm,
             alpha_t, beta_t, idx_in_t, idx_out_t, ex_t, rows_t, rows_a_t,
             u_s, gsem, ssem, iisem, iosem):
    c = lax.axis_index("c")
    s = lax.axis_index("s")

    pltpu.sync_copy(ab_hbm.at[0], alpha_t)
    pltpu.sync_copy(ab_hbm.at[1], beta_t)

    eic = edges_hbm.at[c].at[s]
    eoc = edges_hbm.at[1 - c].at[s]

    zeros16 = jnp.zeros((16,), jnp.float32)
    lane0 = lax.iota(jnp.int32, 16) == 0

    def _zero_row(r, carry):
        for k in range(DW // 16):
            rows_a_t[0, r, pl.ds(16 * k, 16)] = zeros16
        return carry

    def _zero_u():
        lax.fori_loop(0, B, _zero_row, 0)
        for i in range(NCHUNK // B):
            pltpu.sync_copy(rows_a_t.at[0],
                            u_s.at[pl.ds(s * NCHUNK + i * B, B)])

    _zero_u()
    plsc.subcore_barrier()

    def _phase(p, carry0):
        qt = qh_hbm.at[p]
        pltpu.sync_copy(eic.at[pl.ds(0, CH)], idx_in_t.at[0])
        pltpu.sync_copy(eoc.at[pl.ds(0, CH)], idx_out_t.at[0])
        for jj2 in (0, 1):
            pltpu.async_copy(qt.at[idx_in_t.at[0].at[jj2]], rows_t.at[jj2],
                             gsem.at[jj2])

        def _chunk(g, carry):
            sub = lax.rem(g, 2)
            nxt = 1 - sub

            @pl.when(g > 0)
            def _():
                pltpu.make_async_copy(eic.at[pl.ds(g * CH, CH)],
                                      idx_in_t.at[sub], iisem).wait()
                pltpu.make_async_copy(eoc.at[pl.ds(g * CH, CH)],
                                      idx_out_t.at[sub], iosem).wait()
                for jj2 in (0, 1):
                    pltpu.async_copy(qt.at[idx_in_t.at[sub].at[jj2]],
                                     rows_t.at[jj2], gsem.at[jj2])

            @pl.when(g + 1 < NCK)
            def _():
                pltpu.async_copy(eic.at[pl.ds((g + 1) * CH, CH)],
                                 idx_in_t.at[nxt], iisem)
                pltpu.async_copy(eoc.at[pl.ds((g + 1) * CH, CH)],
                                 idx_out_t.at[nxt], iosem)

            for jj in range(CH):
                buf = jj % 2
                pltpu.make_async_copy(qt.at[idx_in_t.at[sub].at[jj]],
                                      rows_t.at[buf], gsem.at[buf]).wait()
                for k in range(B // 16):
                    vi = idx_in_t[sub, jj, pl.ds(16 * k, 16)]
                    vo = idx_out_t[sub, jj, pl.ds(16 * k, 16)]
                    a = plsc.load_gather(alpha_t, [vi])
                    b = plsc.load_gather(beta_t, [vo])
                    sg = 1.0 / (1.0 + jnp.exp(-(a + b)))
                    ex_t[pl.ds(16 * k, 16)] = jnp.exp(sg)

                def _drain_scatter():
                    pltpu.make_async_copy(
                        rows_a_t.at[buf],
                        u_s.at[idx_out_t.at[sub].at[jj]],
                        ssem.at[buf]).wait()
                if jj >= 2:
                    _drain_scatter()
                else:
                    @pl.when(g > 0)
                    def _():
                        _drain_scatter()

                @plsc.parallel_loop(0, B, step=1, unroll=4)
                def _scale_row(r):
                    wv = plsc.load_gather(
                        ex_t, [jnp.full((16,), r, jnp.int32)])
                    for k in range(DH // 32):
                        x32 = rows_t[buf, r, pl.ds(32 * k, 32)]
                        lo, hi = plsc.unpack(x32, format=plsc.PackFormat.INTERLEAVED)
                        rows_a_t[buf, r, pl.ds(32 * k, 16)] = lo * wv
                        rows_a_t[buf, r, pl.ds(32 * k + 16, 16)] = hi * wv
                    rows_a_t[buf, r, pl.ds(DH, 16)] = jnp.where(
                        lane0, wv, zeros16)

                pltpu.async_copy(rows_a_t.at[buf],
                                 u_s.at[idx_out_t.at[sub].at[jj]],
                                 ssem.at[buf], add=True)
                if jj < CH - 2:
                    pltpu.async_copy(qt.at[idx_in_t.at[sub].at[jj + 2]],
                                     rows_t.at[buf], gsem.at[buf])
            return carry

        lax.fori_loop(0, NCK, _chunk, 0)

        for buf in (0, 1):
            pltpu.make_async_copy(rows_a_t.at[buf],
                                  u_s.at[idx_out_t.at[1].at[CH - 2 + buf]],
                                  ssem.at[buf]).wait()
        plsc.subcore_barrier()

        for i in range(NCHUNK // B):
            off = s * NCHUNK + i * B
            pltpu.sync_copy(u_s.at[pl.ds(off, B)],
                            out_hbm.at[c].at[p].at[pl.ds(off, B)])

        @pl.when(p == 0)
        def _():
            _zero_u()
        plsc.subcore_barrier()
        return carry0

    lax.fori_loop(0, 2, _phase, 0)


_sc_call = pl.kernel(
    _sc_body,
    out_type=jax.ShapeDtypeStruct((2, 2, NPAD, DW), jnp.float32),
    mesh=plsc.VectorSubcoreMesh(core_axis_name="c", subcore_axis_name="s",
                                num_cores=2, num_subcores=16),
    compiler_params=pltpu.CompilerParams(needs_layout_passes=False,
                                         use_tc_tiling_on_sc=False),
    scratch_types=[
        pltpu.VMEM((NPAD,), jnp.float32),
        pltpu.VMEM((NPAD,), jnp.float32),
        pltpu.VMEM((2, CH, B), jnp.int32),
        pltpu.VMEM((2, CH, B), jnp.int32),
        pltpu.VMEM((B,), jnp.float32),
        pltpu.VMEM((2, B, DH), jnp.bfloat16),
        pltpu.VMEM((2, B, DW), jnp.float32),
        pltpu.VMEM_SHARED((NPAD, DW), jnp.float32),
        pltpu.SemaphoreType.DMA((2,)),
        pltpu.SemaphoreType.DMA((2,)),
        pltpu.SemaphoreType.DMA,
        pltpu.SemaphoreType.DMA,
    ],
)


def _out_body(q_ref, a00_ref, a01_ref, a10_ref, a11_ref,
              wu_ref, wv_ref, bv_ref, o1_ref, o2_ref):
    a00 = a00_ref[...]
    a01 = a01_ref[...]
    a10 = a10_ref[...]
    a11 = a11_ref[...]
    u1 = jnp.concatenate([a00[:, :DH], a01[:, :DH]], axis=1)
    u2 = jnp.concatenate([a10[:, :DH], a11[:, :DH]], axis=1)
    s1 = a00[:, DH:DH + 1]
    s2 = a10[:, DH:DH + 1]
    inv1 = jnp.where(s1 > 0.0, 1.0 / s1, 0.0)
    inv2 = jnp.where(s2 > 0.0, 1.0 / s2, 0.0)
    ok = u1 * inv1 + u2 * inv2
    q = q_ref[...]
    o1_ref[...] = jnp.dot(2.0 * q + ok, wu_ref[...],
                          preferred_element_type=jnp.float32)
    cnt = ((s1 > 0.0).astype(jnp.float32) + (s2 > 0.0).astype(jnp.float32))
    o2_ref[...] = (jnp.dot(ok, wv_ref[...],
                           preferred_element_type=jnp.float32)
                   + cnt * bv_ref[...])


def _out_call(q, u, W_upd, W_V, bv):
    R = 1000
    grid = (N // R,)
    row_spec = pl.BlockSpec((R, D), lambda i: (i, 0))
    acc_spec = pl.BlockSpec((R, DW), lambda i: (i, 0))
    full_spec = pl.BlockSpec((D, D), lambda i: (0, 0))
    bv_spec = pl.BlockSpec((1, D), lambda i: (0, 0))
    return pl.pallas_call(
        _out_body,
        grid=grid,
        in_specs=[row_spec, acc_spec, acc_spec, acc_spec, acc_spec,
                  full_spec, full_spec, bv_spec],
        out_specs=[row_spec, row_spec],
        out_shape=[jax.ShapeDtypeStruct((N, D), jnp.float32),
                   jax.ShapeDtypeStruct((N, D), jnp.float32)],
    )(q, u[0, 0], u[0, 1], u[1, 0], u[1, 1], W_upd, W_V, bv)


def _make_qperm():
    perm = []
    for blk in range(D // 32):
        base = blk * 32
        for i in range(16):
            perm.append(base + i)
            perm.append(base + 16 + i)
    return tuple(perm)


_QPERM = _make_qperm()


def _make_qperm_mat():
    import numpy as _np
    m = _np.zeros((D, D), dtype=_np.float32)
    for j, orig in enumerate(_QPERM):
        m[orig, j] = 1.0
    return m


_QPERM_MAT = _make_qperm_mat()


@jax.jit
def kernel(q, edge_index, W_att, W_upd, W_V, b_V):
    src = edge_index[0]
    dst = edge_index[1]
    pads = EPAD - E
    srcp = jnp.pad(src, (0, pads), constant_values=N).reshape(16, NB, B)
    dstp = jnp.pad(dst, (0, pads), constant_values=N).reshape(16, NB, B)
    edges3 = jnp.stack([srcp, dstp])
    qpad = jnp.pad(q, ((0, NPAD - N), (0, 0)))
    w2 = jnp.stack([W_att[:D, 0], W_att[D:, 0]], axis=1)

    pm = jnp.asarray(_QPERM_MAT)
    ab2, qh = _prep_call(qpad, w2, pm)
    u = _sc_call(qh, edges3, ab2.T)
    o1, o2 = _out_call(q, u, W_upd, W_V, b_V.reshape(1, D))
    return o1, o2


# speedup vs baseline: 3.3332x; 1.3334x over previous
import jax
import jax.numpy as jnp
from jax import lax
from jax.experimental import pallas as pl
from jax.experimental.pallas import tpu as pltpu
from jax.experimental.pallas import tpu_sc as plsc

N = 10000
D = 128
E = 320000

NPAD = 10240
B = 128
NB = 160
CH = 8
NCK = NB // CH
ECH = NB * B
EPAD = 16 * ECH
NCHUNK = NPAD // 16
DH = D // 2
DW = DH + 16


def _prep_body(q_ref, w2_ref, pm_ref, ab_ref, qh_ref):
    q = q_ref[...]
    ab_ref[...] = jnp.dot(q, w2_ref[...], preferred_element_type=jnp.float32)
    qp = jnp.dot(q, pm_ref[...], preferred_element_type=jnp.float32)
    qb = qp.astype(jnp.bfloat16)
    qh_ref[0] = qb[:, :DH]
    qh_ref[1] = qb[:, DH:]


def _prep_call(qpad, w2, pm):
    R = 1024
    return pl.pallas_call(
        _prep_body,
        grid=(NPAD // R,),
        in_specs=[pl.BlockSpec((R, D), lambda i: (i, 0)),
                  pl.BlockSpec((D, 2), lambda i: (0, 0)),
                  pl.BlockSpec((D, D), lambda i: (0, 0))],
        out_specs=[pl.BlockSpec((R, 2), lambda i: (i, 0)),
                   pl.BlockSpec((2, R, DH), lambda i: (0, i, 0))],
        out_shape=[jax.ShapeDtypeStruct((NPAD, 2), jnp.float32),
                   jax.ShapeDtypeStruct((2, NPAD, DH), jnp.bfloat16)],
    )(qpad, w2, pm)


def _sc_body(qh_hbm, edges_hbm, ab_hbm, out_hbm,
             alpha_t, beta_t, idx_in_t, idx_out_t, ex_t, rows_t, rows_a_t,
             u_s, qs_s, gsem, ssem, iisem, iosem):
    c = lax.axis_index("c")
    s = lax.axis_index("s")

    pltpu.sync_copy(ab_hbm.at[0], alpha_t)
    pltpu.sync_copy(ab_hbm.at[1], beta_t)

    eic = edges_hbm.at[c].at[s]
    eoc = edges_hbm.at[1 - c].at[s]

    zeros16 = jnp.zeros((16,), jnp.float32)
    lane0 = lax.iota(jnp.int32, 16) == 0

    def _zero_row(r, carry):
        for k in range(DW // 16):
            rows_a_t[0, r, pl.ds(16 * k, 16)] = zeros16
        return carry

    def _zero_u():
        lax.fori_loop(0, B, _zero_row, 0)
        for i in range(NCHUNK // B):
            pltpu.sync_copy(rows_a_t.at[0],
                            u_s.at[pl.ds(s * NCHUNK + i * B, B)])

    def _phase(p, carry0):
        pltpu.sync_copy(qh_hbm.at[p].at[pl.ds(s * NCHUNK, NCHUNK)],
                        qs_s.at[pl.ds(s * NCHUNK, NCHUNK)])
        _zero_u()
        plsc.subcore_barrier()
        qt = qs_s
        pltpu.sync_copy(eic.at[pl.ds(0, CH)], idx_in_t.at[0])
        pltpu.sync_copy(eoc.at[pl.ds(0, CH)], idx_out_t.at[0])
        for jj2 in (0, 1):
            pltpu.async_copy(qt.at[idx_in_t.at[0].at[jj2]], rows_t.at[jj2],
                             gsem.at[jj2])

        def _chunk(g, carry):
            sub = lax.rem(g, 2)
            nxt = 1 - sub

            @pl.when(g > 0)
            def _():
                pltpu.make_async_copy(eic.at[pl.ds(g * CH, CH)],
                                      idx_in_t.at[sub], iisem).wait()
                pltpu.make_async_copy(eoc.at[pl.ds(g * CH, CH)],
                                      idx_out_t.at[sub], iosem).wait()
                for jj2 in (0, 1):
                    pltpu.async_copy(qt.at[idx_in_t.at[sub].at[jj2]],
                                     rows_t.at[jj2], gsem.at[jj2])

            @pl.when(g + 1 < NCK)
            def _():
                pltpu.async_copy(eic.at[pl.ds((g + 1) * CH, CH)],
                                 idx_in_t.at[nxt], iisem)
                pltpu.async_copy(eoc.at[pl.ds((g + 1) * CH, CH)],
                                 idx_out_t.at[nxt], iosem)

            for jj in range(CH):
                buf = jj % 2
                pltpu.make_async_copy(qt.at[idx_in_t.at[sub].at[jj]],
                                      rows_t.at[buf], gsem.at[buf]).wait()
                for k in range(B // 16):
                    vi = idx_in_t[sub, jj, pl.ds(16 * k, 16)]
                    vo = idx_out_t[sub, jj, pl.ds(16 * k, 16)]
                    a = plsc.load_gather(alpha_t, [vi])
                    b = plsc.load_gather(beta_t, [vo])
                    sg = 1.0 / (1.0 + jnp.exp(-(a + b)))
                    ex_t[pl.ds(16 * k, 16)] = jnp.exp(sg)

                def _drain_scatter():
                    pltpu.make_async_copy(
                        rows_a_t.at[buf],
                        u_s.at[idx_out_t.at[sub].at[jj]],
                        ssem.at[buf]).wait()
                if jj >= 2:
                    _drain_scatter()
                else:
                    @pl.when(g > 0)
                    def _():
                        _drain_scatter()

                @plsc.parallel_loop(0, B, step=1, unroll=4)
                def _scale_row(r):
                    wv = plsc.load_gather(
                        ex_t, [jnp.full((16,), r, jnp.int32)])
                    for k in range(DH // 32):
                        x32 = rows_t[buf, r, pl.ds(32 * k, 32)]
                        lo, hi = plsc.unpack(x32, format=plsc.PackFormat.INTERLEAVED)
                        rows_a_t[buf, r, pl.ds(32 * k, 16)] = lo * wv
                        rows_a_t[buf, r, pl.ds(32 * k + 16, 16)] = hi * wv
                    rows_a_t[buf, r, pl.ds(DH, 16)] = jnp.where(
                        lane0, wv, zeros16)

                pltpu.async_copy(rows_a_t.at[buf],
                                 u_s.at[idx_out_t.at[sub].at[jj]],
                                 ssem.at[buf], add=True)
                if jj < CH - 2:
                    pltpu.async_copy(qt.at[idx_in_t.at[sub].at[jj + 2]],
                                     rows_t.at[buf], gsem.at[buf])
            return carry

        lax.fori_loop(0, NCK, _chunk, 0)

        for buf in (0, 1):
            pltpu.make_async_copy(rows_a_t.at[buf],
                                  u_s.at[idx_out_t.at[1].at[CH - 2 + buf]],
                                  ssem.at[buf]).wait()
        plsc.subcore_barrier()

        for i in range(NCHUNK // B):
            off = s * NCHUNK + i * B
            pltpu.sync_copy(u_s.at[pl.ds(off, B)],
                            out_hbm.at[c].at[p].at[pl.ds(off, B)])

        return carry0

    lax.fori_loop(0, 2, _phase, 0)


_sc_call = pl.kernel(
    _sc_body,
    out_type=jax.ShapeDtypeStruct((2, 2, NPAD, DW), jnp.float32),
    mesh=plsc.VectorSubcoreMesh(core_axis_name="c", subcore_axis_name="s",
                                num_cores=2, num_subcores=16),
    compiler_params=pltpu.CompilerParams(needs_layout_passes=False,
                                         use_tc_tiling_on_sc=False),
    scratch_types=[
        pltpu.VMEM((NPAD,), jnp.float32),
        pltpu.VMEM((NPAD,), jnp.float32),
        pltpu.VMEM((2, CH, B), jnp.int32),
        pltpu.VMEM((2, CH, B), jnp.int32),
        pltpu.VMEM((B,), jnp.float32),
        pltpu.VMEM((2, B, DH), jnp.bfloat16),
        pltpu.VMEM((2, B, DW), jnp.float32),
        pltpu.VMEM_SHARED((NPAD, DW), jnp.float32),
        pltpu.VMEM_SHARED((NPAD, DH), jnp.bfloat16),
        pltpu.SemaphoreType.DMA((2,)),
        pltpu.SemaphoreType.DMA((2,)),
        pltpu.SemaphoreType.DMA,
        pltpu.SemaphoreType.DMA,
    ],
)


def _out_body(q_ref, a00_ref, a01_ref, a10_ref, a11_ref,
              wu_ref, wv_ref, bv_ref, o1_ref, o2_ref):
    a00 = a00_ref[...]
    a01 = a01_ref[...]
    a10 = a10_ref[...]
    a11 = a11_ref[...]
    u1 = jnp.concatenate([a00[:, :DH], a01[:, :DH]], axis=1)
    u2 = jnp.concatenate([a10[:, :DH], a11[:, :DH]], axis=1)
    s1 = a00[:, DH:DH + 1]
    s2 = a10[:, DH:DH + 1]
    inv1 = jnp.where(s1 > 0.0, 1.0 / s1, 0.0)
    inv2 = jnp.where(s2 > 0.0, 1.0 / s2, 0.0)
    ok = u1 * inv1 + u2 * inv2
    q = q_ref[...]
    o1_ref[...] = jnp.dot(2.0 * q + ok, wu_ref[...],
                          preferred_element_type=jnp.float32)
    cnt = ((s1 > 0.0).astype(jnp.float32) + (s2 > 0.0).astype(jnp.float32))
    o2_ref[...] = (jnp.dot(ok, wv_ref[...],
                           preferred_element_type=jnp.float32)
                   + cnt * bv_ref[...])


def _out_call(q, u, W_upd, W_V, bv):
    R = 1000
    grid = (N // R,)
    row_spec = pl.BlockSpec((R, D), lambda i: (i, 0))
    acc_spec = pl.BlockSpec((R, DW), lambda i: (i, 0))
    full_spec = pl.BlockSpec((D, D), lambda i: (0, 0))
    bv_spec = pl.BlockSpec((1, D), lambda i: (0, 0))
    return pl.pallas_call(
        _out_body,
        grid=grid,
        in_specs=[row_spec, acc_spec, acc_spec, acc_spec, acc_spec,
                  full_spec, full_spec, bv_spec],
        out_specs=[row_spec, row_spec],
        out_shape=[jax.ShapeDtypeStruct((N, D), jnp.float32),
                   jax.ShapeDtypeStruct((N, D), jnp.float32)],
    )(q, u[0, 0], u[0, 1], u[1, 0], u[1, 1], W_upd, W_V, bv)


def _make_qperm():
    perm = []
    for blk in range(D // 32):
        base = blk * 32
        for i in range(16):
            perm.append(base + i)
            perm.append(base + 16 + i)
    return tuple(perm)


_QPERM = _make_qperm()


def _make_qperm_mat():
    import numpy as _np
    m = _np.zeros((D, D), dtype=_np.float32)
    for j, orig in enumerate(_QPERM):
        m[orig, j] = 1.0
    return m


_QPERM_MAT = _make_qperm_mat()


@jax.jit
def kernel(q, edge_index, W_att, W_upd, W_V, b_V):
    src = edge_index[0]
    dst = edge_index[1]
    pads = EPAD - E
    srcp = jnp.pad(src, (0, pads), constant_values=N).reshape(16, NB, B)
    dstp = jnp.pad(dst, (0, pads), constant_values=N).reshape(16, NB, B)
    edges3 = jnp.stack([srcp, dstp])
    qpad = jnp.pad(q, ((0, NPAD - N), (0, 0)))
    w2 = jnp.stack([W_att[:D, 0], W_att[D:, 0]], axis=1)

    pm = jnp.asarray(_QPERM_MAT)
    ab2, qh = _prep_call(qpad, w2, pm)
    u = _sc_call(qh, edges3, ab2.T)
    o1, o2 = _out_call(q, u, W_upd, W_V, b_V.reshape(1, D))
    return o1, o2
